# Initial kernel scaffold; baseline (speedup 1.0000x reference)
import jax
import jax.numpy as jnp
from jax.experimental import pallas as pl


def kernel(x, edge_index, edge_attr, batch, W1, root1, b1, W2, root2, b2, Wd, bd):
    raise NotImplementedError("write your pallas kernel here")


# speedup vs baseline: 22.4851x; 22.4851x over previous
import functools

import jax
import jax.numpy as jnp
from jax import lax
from jax.experimental import pallas as pl
from jax.experimental.pallas import tpu as pltpu
from jax.experimental.pallas import tpu_sc as plsc

N = 10000
E = 320000
REL = 8
G = 64
D_IN = 128
D_H1 = 32
D_H2 = 16

NC = 2
NS = 16
NW = NC * NS
C = 128
CHUNKS_PER_W = (E + NW * C - 1) // (NW * C)
E_PAD = NW * C * CHUNKS_PER_W
EW = CHUNKS_PER_W * C

KEYS = N * REL
KEYS_PAD = 80128
KEYS_PER_S = KEYS_PAD // NS
ACC_ROWS = 10240
ROWS_PER_S = ACC_ROWS // NS

_mesh = plsc.VectorSubcoreMesh(core_axis_name="c", subcore_axis_name="s")


def _worker(c, s):
    return s * NC + c


_sc_params = pltpu.CompilerParams(use_tc_tiling_on_sc=False)


@functools.partial(
    pl.kernel,
    out_type=jax.ShapeDtypeStruct((NC, NS, KEYS_PER_S), jnp.float32),
    mesh=_mesh,
    compiler_params=_sc_params,
    scratch_types=[
        pltpu.VMEM((C,), jnp.int32),
        pltpu.VMEM((C,), jnp.int32),
        pltpu.VMEM((C,), jnp.int32),
        pltpu.VMEM((C,), jnp.float32),
        pltpu.VMEM((KEYS_PER_S,), jnp.float32),
        pltpu.VMEM_SHARED((KEYS_PAD,), jnp.float32),
    ],
)
def _count_sc(dst_hbm, et_hbm, cnt_out, dst_v, et_v, key_v, ones_v, stage_v, cnt_sh):
    c = lax.axis_index("c")
    s = lax.axis_index("s")
    w = _worker(c, s)
    base = w * EW

    for k in range(C // 16):
        sl = pl.ds(k * 16, 16)
        ones_v[sl] = jnp.full((16,), 1.0, jnp.float32)

    def zero_body(i, _):
        stage_v[pl.ds(i * 16, 16)] = jnp.zeros((16,), jnp.float32)
        return 0

    lax.fori_loop(0, KEYS_PER_S // 16, zero_body, 0)
    pltpu.sync_copy(stage_v, cnt_sh.at[pl.ds(s * KEYS_PER_S, KEYS_PER_S)])
    plsc.subcore_barrier()

    def chunk_body(i, _):
        off = base + i * C
        pltpu.sync_copy(dst_hbm.at[pl.ds(off, C)], dst_v)
        pltpu.sync_copy(et_hbm.at[pl.ds(off, C)], et_v)
        for k in range(C // 16):
            sl = pl.ds(k * 16, 16)
            key_v[sl] = dst_v[sl] * REL + et_v[sl]
        pltpu.sync_copy(ones_v, cnt_sh.at[key_v], add=True)
        return 0

    lax.fori_loop(0, CHUNKS_PER_W, chunk_body, 0)
    plsc.subcore_barrier()

    pltpu.sync_copy(cnt_sh.at[pl.ds(s * KEYS_PER_S, KEYS_PER_S)], stage_v)
    pltpu.sync_copy(stage_v, cnt_out.at[c, s])


def _make_agg_sc(D):
    @functools.partial(
        pl.kernel,
        out_type=jax.ShapeDtypeStruct((NC, NS, ROWS_PER_S, D), jnp.float32),
        mesh=_mesh,
        compiler_params=_sc_params,
        scratch_types=[
            pltpu.VMEM((C,), jnp.int32),
            pltpu.VMEM((C,), jnp.int32),
            pltpu.VMEM((C,), jnp.int32),
            pltpu.VMEM((C,), jnp.int32),
            pltpu.VMEM((C,), jnp.int32),
            pltpu.VMEM((C, D), jnp.float32),
            pltpu.VMEM((C,), jnp.float32),
            pltpu.VMEM((ROWS_PER_S, D), jnp.float32),
            pltpu.VMEM_SHARED((ACC_ROWS, D), jnp.float32),
        ],
    )
    def _agg_sc(src_hbm, dst_hbm, et_hbm, xr_hbm, norm_hbm, agg_out,
                src_v, dst_v, et_v, key_v, nkey_v, msg_v, nrm_v, stage_v, acc_sh):
        c = lax.axis_index("c")
        s = lax.axis_index("s")
        w = _worker(c, s)
        base = w * EW

        def zero_body(i, _):
            for j in range(D // 16):
                stage_v[i, pl.ds(j * 16, 16)] = jnp.zeros((16,), jnp.float32)
            return 0

        lax.fori_loop(0, ROWS_PER_S, zero_body, 0)
        pltpu.sync_copy(stage_v, acc_sh.at[pl.ds(s * ROWS_PER_S, ROWS_PER_S)])
        plsc.subcore_barrier()

        def chunk_body(i, _):
            off = base + i * C
            pltpu.sync_copy(src_hbm.at[pl.ds(off, C)], src_v)
            pltpu.sync_copy(dst_hbm.at[pl.ds(off, C)], dst_v)
            pltpu.sync_copy(et_hbm.at[pl.ds(off, C)], et_v)
            for k in range(C // 16):
                sl = pl.ds(k * 16, 16)
                et = et_v[sl]
                key_v[sl] = src_v[sl] * REL + et
                nkey_v[sl] = dst_v[sl] * REL + et
            pltpu.sync_copy(xr_hbm.at[key_v], msg_v)
            pltpu.sync_copy(norm_hbm.at[nkey_v], nrm_v)

            def scale_body(g, _):
                nv16 = nrm_v[pl.ds(g * 16, 16)]
                for t in range(16):
                    nv = nv16[t]
                    e = g * 16 + t
                    for j in range(D // 16):
                        sl = pl.ds(j * 16, 16)
                        msg_v[e, sl] = msg_v[e, sl] * nv
                return 0

            lax.fori_loop(0, C // 16, scale_body, 0)
            pltpu.sync_copy(msg_v, acc_sh.at[dst_v], add=True)
            return 0

        lax.fori_loop(0, CHUNKS_PER_W, chunk_body, 0)
        plsc.subcore_barrier()

        pltpu.sync_copy(acc_sh.at[pl.ds(s * ROWS_PER_S, ROWS_PER_S)], stage_v)
        pltpu.sync_copy(stage_v, agg_out.at[c, s])

    return _agg_sc


_agg32_sc = _make_agg_sc(D_H1)
_agg16_sc = _make_agg_sc(D_H2)


_BN = 1000


def _dense1_tc(x_ref, w_ref, root_ref, b_ref, xr_ref, r1_ref):
    xb = x_ref[...]
    xr_ref[...] = jnp.dot(xb, w_ref[...], preferred_element_type=jnp.float32)
    r1_ref[...] = jnp.dot(xb, root_ref[...], preferred_element_type=jnp.float32) + b_ref[...]


def _norm_tc(cnt_ref, norm_ref):
    cnt = cnt_ref[0:1, :] + cnt_ref[1:2, :]
    norm_ref[...] = jnp.where(cnt > 0, 1.0 / jnp.maximum(cnt, 1.0), 0.0)


def _dense2_tc(r1_ref, agg_ref, w_ref, root_ref, b_ref, xr_ref, r2_ref):
    h1 = jax.nn.relu(r1_ref[...] + agg_ref[0] + agg_ref[1])
    xr_ref[...] = jnp.dot(h1, w_ref[...], preferred_element_type=jnp.float32)
    r2_ref[...] = jnp.dot(h1, root_ref[...], preferred_element_type=jnp.float32) + b_ref[...]


_BNF = 1000


def _finale_tc(r2_ref, agg_ref, batch_ref, wd_ref, bd_ref, out_ref, pooled_acc):
    i = pl.program_id(0)
    h2 = jax.nn.relu(r2_ref[...] + agg_ref[0] + agg_ref[1])
    gid = lax.broadcasted_iota(jnp.int32, (_BNF, G), 1)
    mask = batch_ref[...] == gid

    @pl.when(i == 0)
    def _():
        pooled_acc[...] = jnp.zeros((D_H2, G), jnp.float32)

    for col in range(D_H2):
        m = jnp.where(mask, h2[:, col:col + 1], 0.0)
        part = jnp.max(m, axis=0, keepdims=True)
        pooled_acc[col:col + 1, :] = jnp.maximum(pooled_acc[col:col + 1, :], part)

    @pl.when(i == pl.num_programs(0) - 1)
    def _():
        out_ref[...] = (
            jnp.sum(pooled_acc[...] * wd_ref[...], axis=0, keepdims=True)
            + bd_ref[...]
        )


def kernel(x, edge_index, edge_attr, batch, W1, root1, b1, W2, root2, b2, Wd, bd):
    pad = E_PAD - E
    i32 = jnp.int32
    src_p = jnp.concatenate([edge_index[0].astype(i32), jnp.zeros((pad,), i32)])
    dst_p = jnp.concatenate([edge_index[1].astype(i32), jnp.full((pad,), N, i32)])
    et_p = jnp.concatenate([edge_attr.reshape(-1).astype(i32), jnp.zeros((pad,), i32)])

    w1f = jnp.transpose(W1, (1, 0, 2)).reshape(D_IN, REL * D_H1)
    w2f = jnp.transpose(W2, (1, 0, 2)).reshape(D_H1, REL * D_H2)

    xr1, r1 = pl.pallas_call(
        _dense1_tc,
        grid=(N // _BN,),
        in_specs=[
            pl.BlockSpec((_BN, D_IN), lambda i: (i, 0)),
            pl.BlockSpec((D_IN, REL * D_H1), lambda i: (0, 0)),
            pl.BlockSpec((D_IN, D_H1), lambda i: (0, 0)),
            pl.BlockSpec((1, D_H1), lambda i: (0, 0)),
        ],
        out_specs=[
            pl.BlockSpec((_BN, REL * D_H1), lambda i: (i, 0)),
            pl.BlockSpec((_BN, D_H1), lambda i: (i, 0)),
        ],
        out_shape=[
            jax.ShapeDtypeStruct((N, REL * D_H1), jnp.float32),
            jax.ShapeDtypeStruct((N, D_H1), jnp.float32),
        ],
    )(x, w1f, root1, b1.reshape(1, D_H1))

    cnt = _count_sc(dst_p, et_p).reshape(NC, KEYS_PAD)
    norm = pl.pallas_call(
        _norm_tc,
        in_specs=[pl.BlockSpec((NC, KEYS_PAD), lambda: (0, 0))],
        out_specs=pl.BlockSpec((1, KEYS_PAD), lambda: (0, 0)),
        out_shape=jax.ShapeDtypeStruct((1, KEYS_PAD), jnp.float32),
    )(cnt).reshape(KEYS_PAD)

    agg1 = _agg32_sc(src_p, dst_p, et_p, xr1.reshape(N * REL, D_H1), norm)
    agg1 = agg1.reshape(NC, ACC_ROWS, D_H1)

    xr2, r2 = pl.pallas_call(
        _dense2_tc,
        grid=(N // _BN,),
        in_specs=[
            pl.BlockSpec((_BN, D_H1), lambda i: (i, 0)),
            pl.BlockSpec((NC, _BN, D_H1), lambda i: (0, i, 0)),
            pl.BlockSpec((D_H1, REL * D_H2), lambda i: (0, 0)),
            pl.BlockSpec((D_H1, D_H2), lambda i: (0, 0)),
            pl.BlockSpec((1, D_H2), lambda i: (0, 0)),
        ],
        out_specs=[
            pl.BlockSpec((_BN, REL * D_H2), lambda i: (i, 0)),
            pl.BlockSpec((_BN, D_H2), lambda i: (i, 0)),
        ],
        out_shape=[
            jax.ShapeDtypeStruct((N, REL * D_H2), jnp.float32),
            jax.ShapeDtypeStruct((N, D_H2), jnp.float32),
        ],
    )(r1, agg1, w2f, root2, b2.reshape(1, D_H2))

    agg2 = _agg16_sc(src_p, dst_p, et_p, xr2.reshape(N * REL, D_H2), norm)
    agg2 = agg2.reshape(NC, ACC_ROWS, D_H2)

    out = pl.pallas_call(
        _finale_tc,
        grid=(N // _BNF,),
        in_specs=[
            pl.BlockSpec((_BNF, D_H2), lambda i: (i, 0)),
            pl.BlockSpec((NC, _BNF, D_H2), lambda i: (0, i, 0)),
            pl.BlockSpec((_BNF, 1), lambda i: (i, 0)),
            pl.BlockSpec((D_H2, 1), lambda i: (0, 0)),
            pl.BlockSpec((1, 1), lambda i: (0, 0)),
        ],
        out_specs=pl.BlockSpec((1, G), lambda i: (0, 0)),
        out_shape=jax.ShapeDtypeStruct((1, G), jnp.float32),
        scratch_shapes=[pltpu.VMEM((D_H2, G), jnp.float32)],
    )(r2, agg2, batch.reshape(N, 1).astype(i32), Wd, bd.reshape(1, 1))
    return out.reshape(G, 1)


# speedup vs baseline: 42.3078x; 1.8816x over previous
import functools

import jax
import jax.numpy as jnp
from jax import lax
from jax.experimental import pallas as pl
from jax.experimental.pallas import tpu as pltpu
from jax.experimental.pallas import tpu_sc as plsc

N = 10000
E = 320000
REL = 8
G = 64
D_IN = 128
D_H1 = 32
D_H2 = 16

NC = 2
NS = 16
NW = NC * NS
C = 128
CHUNKS_PER_W = 80
E_PAD = NW * C * CHUNKS_PER_W
EW = CHUNKS_PER_W * C

KEYS = N * REL
KEYS_PAD = 80128
KEYS_PER_S = KEYS_PAD // NS
ACC_ROWS = 10240
ROWS_PER_S = ACC_ROWS // NS

_mesh = plsc.VectorSubcoreMesh(core_axis_name="c", subcore_axis_name="s")


def _worker(c, s):
    return s * NC + c


_sc_params = pltpu.CompilerParams(use_tc_tiling_on_sc=False)


@functools.partial(
    pl.kernel,
    out_type=jax.ShapeDtypeStruct((NC, NS, KEYS_PER_S), jnp.float32),
    mesh=_mesh,
    compiler_params=_sc_params,
    scratch_types=[
        pltpu.VMEM((C,), jnp.int32),
        pltpu.VMEM((C,), jnp.int32),
        pltpu.VMEM((C,), jnp.int32),
        pltpu.VMEM((C,), jnp.float32),
        pltpu.VMEM((KEYS_PER_S,), jnp.float32),
        pltpu.VMEM_SHARED((KEYS_PAD,), jnp.float32),
    ],
)
def _count_sc(dst_hbm, et_hbm, cnt_out, dst_v, et_v, key_v, ones_v, stage_v, cnt_sh):
    c = lax.axis_index("c")
    s = lax.axis_index("s")
    w = _worker(c, s)
    base = w * EW

    for k in range(C // 16):
        sl = pl.ds(k * 16, 16)
        ones_v[sl] = jnp.full((16,), 1.0, jnp.float32)

    def zero_body(i, _):
        stage_v[pl.ds(i * 16, 16)] = jnp.zeros((16,), jnp.float32)
        return 0

    lax.fori_loop(0, KEYS_PER_S // 16, zero_body, 0)
    pltpu.sync_copy(stage_v, cnt_sh.at[pl.ds(s * KEYS_PER_S, KEYS_PER_S)])
    plsc.subcore_barrier()

    def chunk_body(i, _):
        off = base + i * C
        pltpu.sync_copy(dst_hbm.at[pl.ds(off, C)], dst_v)
        pltpu.sync_copy(et_hbm.at[pl.ds(off, C)], et_v)
        for k in range(C // 16):
            sl = pl.ds(k * 16, 16)
            key_v[sl] = dst_v[sl] * REL + et_v[sl]
        pltpu.sync_copy(ones_v, cnt_sh.at[key_v], add=True)
        return 0

    lax.fori_loop(0, CHUNKS_PER_W, chunk_body, 0)
    plsc.subcore_barrier()

    pltpu.sync_copy(cnt_sh.at[pl.ds(s * KEYS_PER_S, KEYS_PER_S)], stage_v)
    pltpu.sync_copy(stage_v, cnt_out.at[c, s])


def _make_agg_sc(D):
    @functools.partial(
        pl.kernel,
        out_type=jax.ShapeDtypeStruct((NC, NS, ROWS_PER_S, D), jnp.float32),
        mesh=_mesh,
        compiler_params=_sc_params,
        scratch_types=[
            [pltpu.VMEM((C,), jnp.int32)] * 3,
            [pltpu.VMEM((C,), jnp.int32)] * 3,
            [pltpu.VMEM((C,), jnp.int32)] * 2,
            [pltpu.VMEM((C,), jnp.int32)] * 2,
            pltpu.VMEM((C,), jnp.int32),
            pltpu.VMEM((C,), jnp.int32),
            pltpu.VMEM((C, D), jnp.float32),
            pltpu.VMEM((C, D), jnp.float32),
            pltpu.VMEM((C,), jnp.float32),
            pltpu.VMEM((C,), jnp.float32),
            pltpu.VMEM((ROWS_PER_S, D), jnp.float32),
            pltpu.VMEM_SHARED((ACC_ROWS, D), jnp.float32),
            [pltpu.SemaphoreType.DMA] * 6,
        ],
    )
    def _agg_sc(src_hbm, dst_hbm, et_hbm, xr_hbm, norm_hbm, agg_out,
                eA, eB, kA, kB, dA, dB, mA, mB, rA, rB, stage_v, acc_sh, sems):
        c = lax.axis_index("c")
        s = lax.axis_index("s")
        w = _worker(c, s)
        base = w * EW
        sem_eA, sem_eB, sem_gA, sem_gB, sem_scA, sem_scB = sems
        hbm = (src_hbm, dst_hbm, et_hbm)

        def zero_body(i, _):
            for j in range(D // 16):
                stage_v[i, pl.ds(j * 16, 16)] = jnp.zeros((16,), jnp.float32)
            return 0

        lax.fori_loop(0, ROWS_PER_S, zero_body, 0)
        pltpu.sync_copy(stage_v, acc_sh.at[pl.ds(s * ROWS_PER_S, ROWS_PER_S)])
        plsc.subcore_barrier()

        def edges_issue(j, e, sem):
            off = base + j * C
            for t in range(3):
                pltpu.async_copy(hbm[t].at[pl.ds(off, C)], e[t], sem)

        def edges_wait(e, sem):
            for t in range(3):
                pltpu.make_async_copy(hbm[t].at[pl.ds(0, C)], e[t], sem).wait()

        def keys(e, k, d):
            for g in range(C // 16):
                sl = pl.ds(g * 16, 16)
                et = e[2][sl]
                dst = e[1][sl]
                k[0][sl] = e[0][sl] * REL + et
                k[1][sl] = dst * REL + et
                d[sl] = dst

        def gathers_issue(k, m, r, sem):
            pltpu.async_copy(xr_hbm.at[k[0]], m, sem)
            pltpu.async_copy(norm_hbm.at[k[1]], r, sem)

        def gathers_wait(k, m, r, sem):
            pltpu.make_async_copy(xr_hbm.at[k[0]], m, sem).wait()
            pltpu.make_async_copy(norm_hbm.at[k[1]], r, sem).wait()

        def scale(m, r):
            def scale_body(g, _):
                nv16 = r[pl.ds(g * 16, 16)]
                for t in range(16):
                    nv = nv16[t]
                    e = g * 16 + t
                    for j in range(D // 16):
                        sl = pl.ds(j * 16, 16)
                        m[e, sl] = m[e, sl] * nv
                return 0

            lax.fori_loop(0, C // 16, scale_body, 0)

        def scat_issue(d, m, sem):
            pltpu.async_copy(m, acc_sh.at[d], sem, add=True)

        def scat_wait(d, m, sem):
            pltpu.make_async_copy(m, acc_sh.at[d], sem).wait()

        edges_issue(0, eA, sem_eA)
        edges_issue(1, eB, sem_eB)
        edges_wait(eA, sem_eA)
        keys(eA, kA, dA)
        gathers_issue(kA, mA, rA, sem_gA)

        def pair_body(jj, _):
            j = jj * 2
            not_last = j + 2 < CHUNKS_PER_W

            @pl.when(j >= 2)
            def _():
                scat_wait(dB, mB, sem_scB)

            @pl.when(not_last)
            def _():
                edges_issue(j + 2, eA, sem_eA)

            edges_wait(eB, sem_eB)
            keys(eB, kB, dB)
            gathers_issue(kB, mB, rB, sem_gB)
            gathers_wait(kA, mA, rA, sem_gA)
            scale(mA, rA)
            scat_issue(dA, mA, sem_scA)

            @pl.when(not_last)
            def _():
                edges_issue(j + 3, eB, sem_eB)
                edges_wait(eA, sem_eA)
                scat_wait(dA, mA, sem_scA)
                keys(eA, kA, dA)
                gathers_issue(kA, mA, rA, sem_gA)

            gathers_wait(kB, mB, rB, sem_gB)
            scale(mB, rB)
            scat_issue(dB, mB, sem_scB)
            return 0

        lax.fori_loop(0, CHUNKS_PER_W // 2, pair_body, 0)
        scat_wait(dA, mA, sem_scA)
        scat_wait(dB, mB, sem_scB)
        plsc.subcore_barrier()

        pltpu.sync_copy(acc_sh.at[pl.ds(s * ROWS_PER_S, ROWS_PER_S)], stage_v)
        pltpu.sync_copy(stage_v, agg_out.at[c, s])

    return _agg_sc


_agg32_sc = _make_agg_sc(D_H1)
_agg16_sc = _make_agg_sc(D_H2)


_BN = 1000


def _dense1_tc(x_ref, w_ref, root_ref, b_ref, xr_ref, r1_ref):
    xb = x_ref[...]
    xr_ref[...] = jnp.dot(xb, w_ref[...], preferred_element_type=jnp.float32)
    r1_ref[...] = jnp.dot(xb, root_ref[...], preferred_element_type=jnp.float32) + b_ref[...]


def _norm_tc(cnt_ref, norm_ref):
    cnt = cnt_ref[0:1, :] + cnt_ref[1:2, :]
    norm_ref[...] = jnp.where(cnt > 0, 1.0 / jnp.maximum(cnt, 1.0), 0.0)


def _dense2_tc(r1_ref, agg_ref, w_ref, root_ref, b_ref, xr_ref, r2_ref):
    h1 = jax.nn.relu(r1_ref[...] + agg_ref[0] + agg_ref[1])
    xr_ref[...] = jnp.dot(h1, w_ref[...], preferred_element_type=jnp.float32)
    r2_ref[...] = jnp.dot(h1, root_ref[...], preferred_element_type=jnp.float32) + b_ref[...]


_BNF = 1000


def _finale_tc(r2_ref, agg_ref, batch_ref, wd_ref, bd_ref, out_ref, pooled_acc):
    i = pl.program_id(0)
    h2 = jax.nn.relu(r2_ref[...] + agg_ref[0] + agg_ref[1])
    gid = lax.broadcasted_iota(jnp.int32, (_BNF, G), 1)
    mask = batch_ref[...] == gid

    @pl.when(i == 0)
    def _():
        pooled_acc[...] = jnp.zeros((D_H2, G), jnp.float32)

    for col in range(D_H2):
        m = jnp.where(mask, h2[:, col:col + 1], 0.0)
        part = jnp.max(m, axis=0, keepdims=True)
        pooled_acc[col:col + 1, :] = jnp.maximum(pooled_acc[col:col + 1, :], part)

    @pl.when(i == pl.num_programs(0) - 1)
    def _():
        out_ref[...] = (
            jnp.sum(pooled_acc[...] * wd_ref[...], axis=0, keepdims=True)
            + bd_ref[...]
        )


def kernel(x, edge_index, edge_attr, batch, W1, root1, b1, W2, root2, b2, Wd, bd):
    pad = E_PAD - E
    i32 = jnp.int32
    src_p = jnp.concatenate([edge_index[0].astype(i32), jnp.zeros((pad,), i32)])
    dst_p = jnp.concatenate([edge_index[1].astype(i32), jnp.full((pad,), N, i32)])
    et_p = jnp.concatenate([edge_attr.reshape(-1).astype(i32), jnp.zeros((pad,), i32)])

    w1f = jnp.transpose(W1, (1, 0, 2)).reshape(D_IN, REL * D_H1)
    w2f = jnp.transpose(W2, (1, 0, 2)).reshape(D_H1, REL * D_H2)

    xr1, r1 = pl.pallas_call(
        _dense1_tc,
        grid=(N // _BN,),
        in_specs=[
            pl.BlockSpec((_BN, D_IN), lambda i: (i, 0)),
            pl.BlockSpec((D_IN, REL * D_H1), lambda i: (0, 0)),
            pl.BlockSpec((D_IN, D_H1), lambda i: (0, 0)),
            pl.BlockSpec((1, D_H1), lambda i: (0, 0)),
        ],
        out_specs=[
            pl.BlockSpec((_BN, REL * D_H1), lambda i: (i, 0)),
            pl.BlockSpec((_BN, D_H1), lambda i: (i, 0)),
        ],
        out_shape=[
            jax.ShapeDtypeStruct((N, REL * D_H1), jnp.float32),
            jax.ShapeDtypeStruct((N, D_H1), jnp.float32),
        ],
    )(x, w1f, root1, b1.reshape(1, D_H1))

    cnt = _count_sc(dst_p, et_p).reshape(NC, KEYS_PAD)
    norm = pl.pallas_call(
        _norm_tc,
        in_specs=[pl.BlockSpec((NC, KEYS_PAD), lambda: (0, 0))],
        out_specs=pl.BlockSpec((1, KEYS_PAD), lambda: (0, 0)),
        out_shape=jax.ShapeDtypeStruct((1, KEYS_PAD), jnp.float32),
    )(cnt).reshape(KEYS_PAD)

    agg1 = _agg32_sc(src_p, dst_p, et_p, xr1.reshape(N * REL, D_H1), norm)
    agg1 = agg1.reshape(NC, ACC_ROWS, D_H1)

    xr2, r2 = pl.pallas_call(
        _dense2_tc,
        grid=(N // _BN,),
        in_specs=[
            pl.BlockSpec((_BN, D_H1), lambda i: (i, 0)),
            pl.BlockSpec((NC, _BN, D_H1), lambda i: (0, i, 0)),
            pl.BlockSpec((D_H1, REL * D_H2), lambda i: (0, 0)),
            pl.BlockSpec((D_H1, D_H2), lambda i: (0, 0)),
            pl.BlockSpec((1, D_H2), lambda i: (0, 0)),
        ],
        out_specs=[
            pl.BlockSpec((_BN, REL * D_H2), lambda i: (i, 0)),
            pl.BlockSpec((_BN, D_H2), lambda i: (i, 0)),
        ],
        out_shape=[
            jax.ShapeDtypeStruct((N, REL * D_H2), jnp.float32),
            jax.ShapeDtypeStruct((N, D_H2), jnp.float32),
        ],
    )(r1, agg1, w2f, root2, b2.reshape(1, D_H2))

    agg2 = _agg16_sc(src_p, dst_p, et_p, xr2.reshape(N * REL, D_H2), norm)
    agg2 = agg2.reshape(NC, ACC_ROWS, D_H2)

    out = pl.pallas_call(
        _finale_tc,
        grid=(N // _BNF,),
        in_specs=[
            pl.BlockSpec((_BNF, D_H2), lambda i: (i, 0)),
            pl.BlockSpec((NC, _BNF, D_H2), lambda i: (0, i, 0)),
            pl.BlockSpec((_BNF, 1), lambda i: (i, 0)),
            pl.BlockSpec((D_H2, 1), lambda i: (0, 0)),
            pl.BlockSpec((1, 1), lambda i: (0, 0)),
        ],
        out_specs=pl.BlockSpec((1, G), lambda i: (0, 0)),
        out_shape=jax.ShapeDtypeStruct((1, G), jnp.float32),
        scratch_shapes=[pltpu.VMEM((D_H2, G), jnp.float32)],
    )(r2, agg2, batch.reshape(N, 1).astype(i32), Wd, bd.reshape(1, 1))
    return out.reshape(G, 1)


# speedup vs baseline: 50.0156x; 1.1822x over previous
import functools

import jax
import jax.numpy as jnp
from jax import lax
from jax.experimental import pallas as pl
from jax.experimental.pallas import tpu as pltpu
from jax.experimental.pallas import tpu_sc as plsc

N = 10000
E = 320000
REL = 8
G = 64
D_IN = 128
D_H1 = 32
D_H2 = 16

NC = 2
NS = 16
NW = NC * NS
C = 128
CHUNKS_PER_W = 80
E_PAD = NW * C * CHUNKS_PER_W
EW = CHUNKS_PER_W * C

KEYS = N * REL
KEYS_PAD = 80128
KEYS_PER_S = KEYS_PAD // NS
ACC_ROWS = 10240
ROWS_PER_S = ACC_ROWS // NS

_mesh = plsc.VectorSubcoreMesh(core_axis_name="c", subcore_axis_name="s")


def _worker(c, s):
    return s * NC + c


_sc_params = pltpu.CompilerParams(use_tc_tiling_on_sc=False)


@functools.partial(
    pl.kernel,
    out_type=jax.ShapeDtypeStruct((NC, NS, KEYS_PER_S), jnp.float32),
    mesh=_mesh,
    compiler_params=_sc_params,
    scratch_types=[
        [pltpu.VMEM((C,), jnp.int32)] * 2,
        [pltpu.VMEM((C,), jnp.int32)] * 2,
        pltpu.VMEM((C,), jnp.int32),
        pltpu.VMEM((C,), jnp.int32),
        pltpu.VMEM((C,), jnp.float32),
        pltpu.VMEM((KEYS_PER_S,), jnp.float32),
        pltpu.VMEM_SHARED((KEYS_PAD,), jnp.float32),
        [pltpu.SemaphoreType.DMA] * 4,
    ],
)
def _count_sc(dst_hbm, et_hbm, cnt_out, eA, eB, kA, kB, ones_v, stage_v, cnt_sh, sems):
    c = lax.axis_index("c")
    s = lax.axis_index("s")
    w = _worker(c, s)
    base = w * EW
    sem_eA, sem_eB, sem_scA, sem_scB = sems
    hbm = (dst_hbm, et_hbm)

    for k in range(C // 16):
        sl = pl.ds(k * 16, 16)
        ones_v[sl] = jnp.full((16,), 1.0, jnp.float32)

    def zero_body(i, _):
        stage_v[pl.ds(i * 16, 16)] = jnp.zeros((16,), jnp.float32)
        return 0

    lax.fori_loop(0, KEYS_PER_S // 16, zero_body, 0)
    pltpu.sync_copy(stage_v, cnt_sh.at[pl.ds(s * KEYS_PER_S, KEYS_PER_S)])
    plsc.subcore_barrier()

    def edges_issue(j, e, sem):
        off = base + j * C
        for t in range(2):
            pltpu.async_copy(hbm[t].at[pl.ds(off, C)], e[t], sem)

    def edges_wait(e, sem):
        for t in range(2):
            pltpu.make_async_copy(hbm[t].at[pl.ds(0, C)], e[t], sem).wait()

    def keys(e, k):
        for g in range(C // 16):
            sl = pl.ds(g * 16, 16)
            k[sl] = e[0][sl] * REL + e[1][sl]

    edges_issue(0, eA, sem_eA)
    edges_issue(1, eB, sem_eB)

    def step(j, e, k, sem_e, sem_sc):
        not_last = j + 2 < CHUNKS_PER_W
        edges_wait(e, sem_e)

        @pl.when(j >= 2)
        def _():
            pltpu.make_async_copy(ones_v, cnt_sh.at[k], sem_sc).wait()

        keys(e, k)

        @pl.when(not_last)
        def _():
            edges_issue(j + 2, e, sem_e)

        pltpu.async_copy(ones_v, cnt_sh.at[k], sem_sc, add=True)

    def pair_body(jj, _):
        j = jj * 2
        step(j, eA, kA, sem_eA, sem_scA)
        step(j + 1, eB, kB, sem_eB, sem_scB)
        return 0

    lax.fori_loop(0, CHUNKS_PER_W // 2, pair_body, 0)
    pltpu.make_async_copy(ones_v, cnt_sh.at[kA], sem_scA).wait()
    pltpu.make_async_copy(ones_v, cnt_sh.at[kB], sem_scB).wait()
    plsc.subcore_barrier()

    pltpu.sync_copy(cnt_sh.at[pl.ds(s * KEYS_PER_S, KEYS_PER_S)], stage_v)
    pltpu.sync_copy(stage_v, cnt_out.at[c, s])


def _make_agg_sc(D):
    @functools.partial(
        pl.kernel,
        out_type=jax.ShapeDtypeStruct((NC, NS, ROWS_PER_S, D), jnp.float32),
        mesh=_mesh,
        compiler_params=_sc_params,
        scratch_types=[
            [pltpu.VMEM((C,), jnp.int32)] * 3,
            [pltpu.VMEM((C,), jnp.int32)] * 3,
            [pltpu.VMEM((C,), jnp.int32)] * 2,
            [pltpu.VMEM((C,), jnp.int32)] * 2,
            pltpu.VMEM((C,), jnp.int32),
            pltpu.VMEM((C,), jnp.int32),
            pltpu.VMEM((C, D), jnp.float32),
            pltpu.VMEM((C, D), jnp.float32),
            pltpu.VMEM((C,), jnp.float32),
            pltpu.VMEM((C,), jnp.float32),
            pltpu.VMEM((ROWS_PER_S, D), jnp.float32),
            pltpu.VMEM_SHARED((ACC_ROWS, D), jnp.float32),
            [pltpu.SemaphoreType.DMA] * 6,
        ],
    )
    def _agg_sc(src_hbm, dst_hbm, et_hbm, xr_hbm, norm_hbm, agg_out,
                eA, eB, kA, kB, dA, dB, mA, mB, rA, rB, stage_v, acc_sh, sems):
        c = lax.axis_index("c")
        s = lax.axis_index("s")
        w = _worker(c, s)
        base = w * EW
        sem_eA, sem_eB, sem_gA, sem_gB, sem_scA, sem_scB = sems
        hbm = (src_hbm, dst_hbm, et_hbm)

        def zero_body(i, _):
            for j in range(D // 16):
                stage_v[i, pl.ds(j * 16, 16)] = jnp.zeros((16,), jnp.float32)
            return 0

        lax.fori_loop(0, ROWS_PER_S, zero_body, 0)
        pltpu.sync_copy(stage_v, acc_sh.at[pl.ds(s * ROWS_PER_S, ROWS_PER_S)])
        plsc.subcore_barrier()

        def edges_issue(j, e, sem):
            off = base + j * C
            for t in range(3):
                pltpu.async_copy(hbm[t].at[pl.ds(off, C)], e[t], sem)

        def edges_wait(e, sem):
            for t in range(3):
                pltpu.make_async_copy(hbm[t].at[pl.ds(0, C)], e[t], sem).wait()

        def keys(e, k, d):
            for g in range(C // 16):
                sl = pl.ds(g * 16, 16)
                et = e[2][sl]
                dst = e[1][sl]
                k[0][sl] = e[0][sl] * REL + et
                k[1][sl] = dst * REL + et
                d[sl] = dst

        def gathers_issue(k, m, r, sem):
            pltpu.async_copy(xr_hbm.at[k[0]], m, sem)
            pltpu.async_copy(norm_hbm.at[k[1]], r, sem)

        def gathers_wait(k, m, r, sem):
            pltpu.make_async_copy(xr_hbm.at[k[0]], m, sem).wait()
            pltpu.make_async_copy(norm_hbm.at[k[1]], r, sem).wait()

        def scale(m, r):
            def scale_body(g, _):
                nv16 = r[pl.ds(g * 16, 16)]
                for t in range(16):
                    nv = nv16[t]
                    e = g * 16 + t
                    for j in range(D // 16):
                        sl = pl.ds(j * 16, 16)
                        m[e, sl] = m[e, sl] * nv
                return 0

            lax.fori_loop(0, C // 16, scale_body, 0)

        def scat_issue(d, m, sem):
            pltpu.async_copy(m, acc_sh.at[d], sem, add=True)

        def scat_wait(d, m, sem):
            pltpu.make_async_copy(m, acc_sh.at[d], sem).wait()

        edges_issue(0, eA, sem_eA)
        edges_issue(1, eB, sem_eB)
        edges_wait(eA, sem_eA)
        keys(eA, kA, dA)
        gathers_issue(kA, mA, rA, sem_gA)

        def pair_body(jj, _):
            j = jj * 2
            not_last = j + 2 < CHUNKS_PER_W

            @pl.when(j >= 2)
            def _():
                scat_wait(dB, mB, sem_scB)

            @pl.when(not_last)
            def _():
                edges_issue(j + 2, eA, sem_eA)

            edges_wait(eB, sem_eB)
            keys(eB, kB, dB)
            gathers_issue(kB, mB, rB, sem_gB)
            gathers_wait(kA, mA, rA, sem_gA)
            scale(mA, rA)
            scat_issue(dA, mA, sem_scA)

            @pl.when(not_last)
            def _():
                edges_issue(j + 3, eB, sem_eB)
                edges_wait(eA, sem_eA)
                scat_wait(dA, mA, sem_scA)
                keys(eA, kA, dA)
                gathers_issue(kA, mA, rA, sem_gA)

            gathers_wait(kB, mB, rB, sem_gB)
            scale(mB, rB)
            scat_issue(dB, mB, sem_scB)
            return 0

        lax.fori_loop(0, CHUNKS_PER_W // 2, pair_body, 0)
        scat_wait(dA, mA, sem_scA)
        scat_wait(dB, mB, sem_scB)
        plsc.subcore_barrier()

        pltpu.sync_copy(acc_sh.at[pl.ds(s * ROWS_PER_S, ROWS_PER_S)], stage_v)
        pltpu.sync_copy(stage_v, agg_out.at[c, s])

    return _agg_sc


_agg32_sc = _make_agg_sc(D_H1)
_agg16_sc = _make_agg_sc(D_H2)


_BN = 1000


def _dense1_tc(x_ref, w_ref, root_ref, b_ref, xr_ref, r1_ref):
    xb = x_ref[...]
    xr_ref[...] = jnp.dot(xb, w_ref[...], preferred_element_type=jnp.float32)
    r1_ref[...] = jnp.dot(xb, root_ref[...], preferred_element_type=jnp.float32) + b_ref[...]


def _norm_tc(cnt_ref, norm_ref):
    cnt = cnt_ref[0:1, :] + cnt_ref[1:2, :]
    norm_ref[...] = jnp.where(cnt > 0, 1.0 / jnp.maximum(cnt, 1.0), 0.0)


def _dense2_tc(r1_ref, agg_ref, w_ref, root_ref, b_ref, xr_ref, r2_ref):
    h1 = jax.nn.relu(r1_ref[...] + agg_ref[0] + agg_ref[1])
    xr_ref[...] = jnp.dot(h1, w_ref[...], preferred_element_type=jnp.float32)
    r2_ref[...] = jnp.dot(h1, root_ref[...], preferred_element_type=jnp.float32) + b_ref[...]


_BNF = 1000


def _finale_tc(r2_ref, agg_ref, batch_ref, wd_ref, bd_ref, out_ref, pooled_acc):
    i = pl.program_id(0)
    h2 = jax.nn.relu(r2_ref[...] + agg_ref[0] + agg_ref[1])
    gid = lax.broadcasted_iota(jnp.int32, (_BNF, G), 1)
    mask = batch_ref[...] == gid

    @pl.when(i == 0)
    def _():
        pooled_acc[...] = jnp.zeros((D_H2, G), jnp.float32)

    for col in range(D_H2):
        m = jnp.where(mask, h2[:, col:col + 1], 0.0)
        part = jnp.max(m, axis=0, keepdims=True)
        pooled_acc[col:col + 1, :] = jnp.maximum(pooled_acc[col:col + 1, :], part)

    @pl.when(i == pl.num_programs(0) - 1)
    def _():
        out_ref[...] = (
            jnp.sum(pooled_acc[...] * wd_ref[...], axis=0, keepdims=True)
            + bd_ref[...]
        )


def kernel(x, edge_index, edge_attr, batch, W1, root1, b1, W2, root2, b2, Wd, bd):
    pad = E_PAD - E
    i32 = jnp.int32
    src_p = jnp.concatenate([edge_index[0].astype(i32), jnp.zeros((pad,), i32)])
    dst_p = jnp.concatenate([edge_index[1].astype(i32), jnp.full((pad,), N, i32)])
    et_p = jnp.concatenate([edge_attr.reshape(-1).astype(i32), jnp.zeros((pad,), i32)])

    w1f = jnp.transpose(W1, (1, 0, 2)).reshape(D_IN, REL * D_H1)
    w2f = jnp.transpose(W2, (1, 0, 2)).reshape(D_H1, REL * D_H2)

    xr1, r1 = pl.pallas_call(
        _dense1_tc,
        grid=(N // _BN,),
        in_specs=[
            pl.BlockSpec((_BN, D_IN), lambda i: (i, 0)),
            pl.BlockSpec((D_IN, REL * D_H1), lambda i: (0, 0)),
            pl.BlockSpec((D_IN, D_H1), lambda i: (0, 0)),
            pl.BlockSpec((1, D_H1), lambda i: (0, 0)),
        ],
        out_specs=[
            pl.BlockSpec((_BN, REL * D_H1), lambda i: (i, 0)),
            pl.BlockSpec((_BN, D_H1), lambda i: (i, 0)),
        ],
        out_shape=[
            jax.ShapeDtypeStruct((N, REL * D_H1), jnp.float32),
            jax.ShapeDtypeStruct((N, D_H1), jnp.float32),
        ],
    )(x, w1f, root1, b1.reshape(1, D_H1))

    cnt = _count_sc(dst_p, et_p).reshape(NC, KEYS_PAD)
    norm = pl.pallas_call(
        _norm_tc,
        in_specs=[pl.BlockSpec((NC, KEYS_PAD), lambda: (0, 0))],
        out_specs=pl.BlockSpec((1, KEYS_PAD), lambda: (0, 0)),
        out_shape=jax.ShapeDtypeStruct((1, KEYS_PAD), jnp.float32),
    )(cnt).reshape(KEYS_PAD)

    agg1 = _agg32_sc(src_p, dst_p, et_p, xr1.reshape(N * REL, D_H1), norm)
    agg1 = agg1.reshape(NC, ACC_ROWS, D_H1)

    xr2, r2 = pl.pallas_call(
        _dense2_tc,
        grid=(N // _BN,),
        in_specs=[
            pl.BlockSpec((_BN, D_H1), lambda i: (i, 0)),
            pl.BlockSpec((NC, _BN, D_H1), lambda i: (0, i, 0)),
            pl.BlockSpec((D_H1, REL * D_H2), lambda i: (0, 0)),
            pl.BlockSpec((D_H1, D_H2), lambda i: (0, 0)),
            pl.BlockSpec((1, D_H2), lambda i: (0, 0)),
        ],
        out_specs=[
            pl.BlockSpec((_BN, REL * D_H2), lambda i: (i, 0)),
            pl.BlockSpec((_BN, D_H2), lambda i: (i, 0)),
        ],
        out_shape=[
            jax.ShapeDtypeStruct((N, REL * D_H2), jnp.float32),
            jax.ShapeDtypeStruct((N, D_H2), jnp.float32),
        ],
    )(r1, agg1, w2f, root2, b2.reshape(1, D_H2))

    agg2 = _agg16_sc(src_p, dst_p, et_p, xr2.reshape(N * REL, D_H2), norm)
    agg2 = agg2.reshape(NC, ACC_ROWS, D_H2)

    out = pl.pallas_call(
        _finale_tc,
        grid=(N // _BNF,),
        in_specs=[
            pl.BlockSpec((_BNF, D_H2), lambda i: (i, 0)),
            pl.BlockSpec((NC, _BNF, D_H2), lambda i: (0, i, 0)),
            pl.BlockSpec((_BNF, 1), lambda i: (i, 0)),
            pl.BlockSpec((D_H2, 1), lambda i: (0, 0)),
            pl.BlockSpec((1, 1), lambda i: (0, 0)),
        ],
        out_specs=pl.BlockSpec((1, G), lambda i: (0, 0)),
        out_shape=jax.ShapeDtypeStruct((1, G), jnp.float32),
        scratch_shapes=[pltpu.VMEM((D_H2, G), jnp.float32)],
    )(r2, agg2, batch.reshape(N, 1).astype(i32), Wd, bd.reshape(1, 1))
    return out.reshape(G, 1)
